# Initial kernel scaffold; baseline (speedup 1.0000x reference)
import functools

import jax
import jax.numpy as jnp
from jax import lax
from jax.experimental import pallas as pl
from jax.experimental.pallas import tpu as pltpu
from jax.experimental.pallas import tpu_sc as plsc

_N = 50000
_B = 16
_NPAD = 51200
_RPT = _NPAD // 16
_CB = 640
_NCB = _RPT // _CB
_E6 = 6 * 800000
_SUP = 2048
_W = 128
_K = _SUP // _W
_NT = 32
_EPAD = ((_E6 + _NT * _SUP - 1) // (_NT * _SUP)) * (_NT * _SUP)
_EPT = _EPAD // _NT
_NSUP = _EPT // _SUP
_F = _NPAD * _B // 128
_MB = 1600


def _step_body(h0_hbm, pprev_hbm, avec_hbm, rows_hbm, cols_hbm, vals_hbm,
               pout_hbm, h_s, acc_s, gath, rows2, cols2, vals1,
               b_h0, b_p0, b_p1, b_h, avv, sem):
    c = lax.axis_index("c")
    s = lax.axis_index("s")
    t = c * 16 + s
    row0 = s * _RPT

    pltpu.sync_copy(avec_hbm, avv)
    av = avv[0, :]
    amv = avv[1, :]
    for cb in range(_NCB):
        base = row0 + cb * _CB
        pltpu.sync_copy(h0_hbm.at[pl.ds(base, _CB)], b_h0)
        pltpu.sync_copy(pprev_hbm.at[0, pl.ds(base, _CB)], b_p0)
        pltpu.sync_copy(pprev_hbm.at[1, pl.ds(base, _CB)], b_p1)

        def blend(i, _):
            b_h[i, :] = b_h0[i, :] * av + (b_p0[i, :] + b_p1[i, :]) * amv
            return 0

        lax.fori_loop(0, _CB, blend, 0, unroll=8)
        pltpu.sync_copy(b_h, h_s.at[pl.ds(base, _CB)])

    def zero(i, _):
        b_h[i, :] = jnp.zeros((16,), jnp.float32)
        return 0

    lax.fori_loop(0, _CB, zero, 0, unroll=8)
    for cb in range(_NCB):
        pltpu.sync_copy(b_h, acc_s.at[pl.ds(row0 + cb * _CB, _CB)])
    plsc.subcore_barrier()

    wbase0 = t * (_EPT // _W)
    ebase0 = t * _EPT

    def super_chunk(i, _):
        pltpu.sync_copy(rows_hbm.at[pl.ds(wbase0 + i * _K, _K)], rows2)
        pltpu.sync_copy(cols_hbm.at[pl.ds(wbase0 + i * _K, _K)], cols2)
        pltpu.sync_copy(vals_hbm.at[pl.ds(ebase0 + i * _SUP, _SUP)], vals1)
        descs = []
        for j in range(_K):
            descs.append(pltpu.async_copy(
                h_s.at[rows2.at[j]], gath.at[pl.ds(j * _W, _W)], sem))
        for d in descs:
            d.wait()

        def scale(e, _):
            gath[e, :] = gath[e, :] * vals1[e]
            return 0

        lax.fori_loop(0, _SUP, scale, 0, unroll=8)
        for j in range(_K):
            pltpu.sync_copy(gath.at[pl.ds(j * _W, _W)],
                            acc_s.at[cols2.at[j]], add=True)
        return 0

    lax.fori_loop(0, _NSUP, super_chunk, 0)
    plsc.subcore_barrier()

    for cb in range(_NCB):
        base = row0 + cb * _CB
        pltpu.sync_copy(acc_s.at[pl.ds(base, _CB)],
                        pout_hbm.at[c, pl.ds(base, _CB)])


_step_call = pl.kernel(
    _step_body,
    out_type=jax.ShapeDtypeStruct((2, _NPAD, _B), jnp.float32),
    mesh=plsc.VectorSubcoreMesh(core_axis_name="c", subcore_axis_name="s"),
    scratch_types=[
        pltpu.VMEM_SHARED((_NPAD, _B), jnp.float32),
        pltpu.VMEM_SHARED((_NPAD, _B), jnp.float32),
        pltpu.VMEM((_SUP, _B), jnp.float32),
        pltpu.VMEM((_K, _W), jnp.int32),
        pltpu.VMEM((_K, _W), jnp.int32),
        pltpu.VMEM((_SUP,), jnp.float32),
        pltpu.VMEM((_CB, _B), jnp.float32),
        pltpu.VMEM((_CB, _B), jnp.float32),
        pltpu.VMEM((_CB, _B), jnp.float32),
        pltpu.VMEM((_CB, _B), jnp.float32),
        pltpu.VMEM((2, 16), jnp.float32),
        pltpu.SemaphoreType.DMA,
    ],
    name="prop_step_sc",
)


def _mlp_body(ctl_ref, u_ref, p0_ref, p1_ref, win_ref, bin_ref, wout_ref,
              par_ref, bias_ref, out_ref):
    a5 = par_ref[0]
    am5 = par_ref[1]
    ctl = ctl_ref[...]
    u = u_ref[...]
    h = u * a5 + (p0_ref[...] + p1_ref[...]) * am5
    acc = jnp.zeros_like(ctl)
    for j in range(64):
        hh = jnp.maximum(
            ctl * win_ref[0, j] + u * win_ref[1, j] + h * win_ref[2, j]
            + bin_ref[j], 0.0)
        acc = acc + hh * wout_ref[j]
    out_ref[...] = acc + bias_ref[...]


_mlp_call = pl.pallas_call(
    _mlp_body,
    out_shape=jax.ShapeDtypeStruct((_F, 128), jnp.float32),
    grid=(_F // _MB,),
    in_specs=[
        pl.BlockSpec((_MB, 128), lambda i: (i, 0)),
        pl.BlockSpec((_MB, 128), lambda i: (i, 0)),
        pl.BlockSpec((_MB, 128), lambda i: (i, 0)),
        pl.BlockSpec((_MB, 128), lambda i: (i, 0)),
        pl.BlockSpec(memory_space=pltpu.SMEM),
        pl.BlockSpec(memory_space=pltpu.SMEM),
        pl.BlockSpec(memory_space=pltpu.SMEM),
        pl.BlockSpec(memory_space=pltpu.SMEM),
        pl.BlockSpec((1, 128), lambda i: (0, 0)),
    ],
    out_specs=pl.BlockSpec((_MB, 128), lambda i: (i, 0)),
    name="mlp_head_tc",
)


def _pad_t(x):
    return jnp.pad(x.T, ((0, _NPAD - _N), (0, 0)))


def kernel(ctl_base, u_raw, cell_idx, rows_tfp, cols_tfp, vals_tfp,
           rows_tfn, cols_tfn, vals_tfn, rows_ppp, cols_ppp, vals_ppp,
           rows_ppn, cols_ppn, vals_ppn, rows_und, cols_und, vals_und,
           rows_mir, cols_mir, vals_mir, g_tf_pos, g_tf_neg, g_ppi_pos,
           g_ppi_neg, g_undir, g_mirna_neg, alpha_logits, cell_emb,
           W_in, b_in, W_out, b_out):
    sp = jax.nn.softplus
    gains = [sp(g_tf_pos), -sp(g_tf_neg), sp(g_ppi_pos), -sp(g_ppi_neg),
             sp(g_undir), -sp(g_mirna_neg)]
    alphas = jax.nn.sigmoid(alpha_logits)

    npad_e = _EPAD - _E6
    pad_idx = (jnp.arange(npad_e, dtype=jnp.int32) % _N)
    rows_all = jnp.concatenate(
        [rows_tfp, rows_tfn, rows_ppp, rows_ppn, rows_und, rows_mir,
         pad_idx]).reshape(_EPAD // _W, _W)
    cols_all = jnp.concatenate(
        [cols_tfp, cols_tfn, cols_ppp, cols_ppn, cols_und, cols_mir,
         pad_idx]).reshape(_EPAD // _W, _W)
    vals_all = jnp.concatenate(
        [g * v for g, v in zip(gains, [vals_tfp, vals_tfn, vals_ppp,
                                       vals_ppn, vals_und, vals_mir])]
        + [jnp.zeros((npad_e,), jnp.float32)])

    h0_t = _pad_t(u_raw)
    ctl_t = _pad_t(ctl_base)

    p = jnp.zeros((2, _NPAD, _B), jnp.float32)
    for k in range(6):
        ab = jnp.float32(1.0) if k == 0 else alphas[k - 1]
        avec = jnp.stack([jnp.full((16,), ab, jnp.float32),
                          jnp.full((16,), 1.0 - ab, jnp.float32)])
        p = _step_call(h0_t, p, avec, rows_all, cols_all, vals_all)

    a5 = alphas[5]
    par = jnp.stack([a5, 1.0 - a5])
    bias_b = cell_emb[cell_idx] @ W_out[:, 0] + b_out[0]
    bias128 = jnp.tile(bias_b, 8)[None, :]
    y_f = _mlp_call(ctl_t.reshape(_F, 128), h0_t.reshape(_F, 128),
                    p[0].reshape(_F, 128), p[1].reshape(_F, 128),
                    W_in, b_in, W_out[:, 0], par, bias128)
    return y_f.reshape(_NPAD, _B)[:_N].T


# speedup vs baseline: 14.7698x; 14.7698x over previous
import functools

import jax
import jax.numpy as jnp
from jax import lax
from jax.experimental import pallas as pl
from jax.experimental.pallas import tpu as pltpu
from jax.experimental.pallas import tpu_sc as plsc

_N = 50000
_B = 16
_NPAD = 51200
_RPT = _NPAD // 16
_CB = 200
_NCB = _RPT // _CB
_E6 = 6 * 800000
_SUP = 1024
_W = 128
_K = _SUP // _W
_NT = 32
_EPAD = ((_E6 + _NT * _SUP - 1) // (_NT * _SUP)) * (_NT * _SUP)
_EPT = _EPAD // _NT
_NSUP = _EPT // _SUP
_F = _NPAD * _B // 128
_MB = 1600


def _step_body(h0_hbm, pprev_hbm, avec_hbm, rows_hbm, cols_hbm, vals_hbm,
               pout_hbm, h_s, acc_s, gath, rows2, cols2, vals1, avv, sem):
    b_h0 = gath.at[pl.ds(0 * _CB, _CB)]
    b_p0 = gath.at[pl.ds(1 * _CB, _CB)]
    b_p1 = gath.at[pl.ds(2 * _CB, _CB)]
    b_h = gath.at[pl.ds(3 * _CB, _CB)]
    c = lax.axis_index("c")
    s = lax.axis_index("s")
    t = c * 16 + s
    row0 = s * _RPT

    pltpu.sync_copy(avec_hbm, avv)
    av = avv[0, :]
    amv = avv[1, :]
    for cb in range(_NCB):
        base = row0 + cb * _CB
        pltpu.sync_copy(h0_hbm.at[pl.ds(base, _CB)], b_h0)
        pltpu.sync_copy(pprev_hbm.at[0, pl.ds(base, _CB)], b_p0)
        pltpu.sync_copy(pprev_hbm.at[1, pl.ds(base, _CB)], b_p1)

        def blend(i, _):
            b_h[i, :] = b_h0[i, :] * av + (b_p0[i, :] + b_p1[i, :]) * amv
            return 0

        lax.fori_loop(0, _CB, blend, 0, unroll=8)
        pltpu.sync_copy(b_h, h_s.at[pl.ds(base, _CB)])

    def zero(i, _):
        b_h[i, :] = jnp.zeros((16,), jnp.float32)
        return 0

    lax.fori_loop(0, _CB, zero, 0, unroll=8)
    for cb in range(_NCB):
        pltpu.sync_copy(b_h, acc_s.at[pl.ds(row0 + cb * _CB, _CB)])
    plsc.subcore_barrier()

    wbase0 = t * (_EPT // _W)
    ebase0 = t * _EPT

    def super_chunk(i, _):
        pltpu.sync_copy(rows_hbm.at[pl.ds(wbase0 + i * _K, _K)], rows2)
        pltpu.sync_copy(cols_hbm.at[pl.ds(wbase0 + i * _K, _K)], cols2)
        pltpu.sync_copy(vals_hbm.at[pl.ds(ebase0 + i * _SUP, _SUP)], vals1)
        descs = []
        for j in range(_K):
            descs.append(pltpu.async_copy(
                h_s.at[rows2.at[j]], gath.at[pl.ds(j * _W, _W)], sem))
        for d in descs:
            d.wait()

        def scale(g, _):
            base = g * 16
            v16 = vals1[pl.ds(base, 16)]
            for l in range(16):
                gath[base + l, :] = gath[base + l, :] * v16[l]
            return 0

        lax.fori_loop(0, _SUP // 16, scale, 0)
        for j in range(_K):
            pltpu.sync_copy(gath.at[pl.ds(j * _W, _W)],
                            acc_s.at[cols2.at[j]], add=True)
        return 0

    lax.fori_loop(0, _NSUP, super_chunk, 0)
    plsc.subcore_barrier()

    pltpu.sync_copy(acc_s.at[pl.ds(row0, _RPT)],
                    pout_hbm.at[c, pl.ds(row0, _RPT)])


_step_call = pl.kernel(
    _step_body,
    out_type=jax.ShapeDtypeStruct((2, _NPAD, _B), jnp.float32),
    mesh=plsc.VectorSubcoreMesh(core_axis_name="c", subcore_axis_name="s"),
    compiler_params=pltpu.CompilerParams(use_tc_tiling_on_sc=False),
    scratch_types=[
        pltpu.VMEM_SHARED((_NPAD, _B), jnp.float32),
        pltpu.VMEM_SHARED((_NPAD, _B), jnp.float32),
        pltpu.VMEM((_SUP, _B), jnp.float32),
        pltpu.VMEM((_K, _W), jnp.int32),
        pltpu.VMEM((_K, _W), jnp.int32),
        pltpu.VMEM((_SUP,), jnp.float32),
        pltpu.VMEM((2, 16), jnp.float32),
        pltpu.SemaphoreType.DMA,
    ],
    name="prop_step_sc",
)


def _mlp_body(ctl_ref, u_ref, p0_ref, p1_ref, win_ref, bin_ref, wout_ref,
              par_ref, bias_ref, out_ref):
    a5 = par_ref[0]
    am5 = par_ref[1]
    ctl = ctl_ref[...]
    u = u_ref[...]
    h = u * a5 + (p0_ref[...] + p1_ref[...]) * am5
    acc = jnp.zeros_like(ctl)
    for j in range(64):
        hh = jnp.maximum(
            ctl * win_ref[0, j] + u * win_ref[1, j] + h * win_ref[2, j]
            + bin_ref[j], 0.0)
        acc = acc + hh * wout_ref[j]
    out_ref[...] = acc + bias_ref[...]


_mlp_call = pl.pallas_call(
    _mlp_body,
    out_shape=jax.ShapeDtypeStruct((_F, 128), jnp.float32),
    grid=(_F // _MB,),
    in_specs=[
        pl.BlockSpec((_MB, 128), lambda i: (i, 0)),
        pl.BlockSpec((_MB, 128), lambda i: (i, 0)),
        pl.BlockSpec((_MB, 128), lambda i: (i, 0)),
        pl.BlockSpec((_MB, 128), lambda i: (i, 0)),
        pl.BlockSpec(memory_space=pltpu.SMEM),
        pl.BlockSpec(memory_space=pltpu.SMEM),
        pl.BlockSpec(memory_space=pltpu.SMEM),
        pl.BlockSpec(memory_space=pltpu.SMEM),
        pl.BlockSpec((1, 128), lambda i: (0, 0)),
    ],
    out_specs=pl.BlockSpec((_MB, 128), lambda i: (i, 0)),
    name="mlp_head_tc",
)


def _pad_t(x):
    return jnp.pad(x.T, ((0, _NPAD - _N), (0, 0)))


def kernel(ctl_base, u_raw, cell_idx, rows_tfp, cols_tfp, vals_tfp,
           rows_tfn, cols_tfn, vals_tfn, rows_ppp, cols_ppp, vals_ppp,
           rows_ppn, cols_ppn, vals_ppn, rows_und, cols_und, vals_und,
           rows_mir, cols_mir, vals_mir, g_tf_pos, g_tf_neg, g_ppi_pos,
           g_ppi_neg, g_undir, g_mirna_neg, alpha_logits, cell_emb,
           W_in, b_in, W_out, b_out):
    sp = jax.nn.softplus
    gains = [sp(g_tf_pos), -sp(g_tf_neg), sp(g_ppi_pos), -sp(g_ppi_neg),
             sp(g_undir), -sp(g_mirna_neg)]
    alphas = jax.nn.sigmoid(alpha_logits)

    npad_e = _EPAD - _E6
    pad_idx = (jnp.arange(npad_e, dtype=jnp.int32) % _N)
    rows_all = jnp.concatenate(
        [rows_tfp, rows_tfn, rows_ppp, rows_ppn, rows_und, rows_mir,
         pad_idx]).reshape(_EPAD // _W, _W)
    cols_all = jnp.concatenate(
        [cols_tfp, cols_tfn, cols_ppp, cols_ppn, cols_und, cols_mir,
         pad_idx]).reshape(_EPAD // _W, _W)
    vals_all = jnp.concatenate(
        [g * v for g, v in zip(gains, [vals_tfp, vals_tfn, vals_ppp,
                                       vals_ppn, vals_und, vals_mir])]
        + [jnp.zeros((npad_e,), jnp.float32)])

    h0_t = _pad_t(u_raw)
    ctl_t = _pad_t(ctl_base)

    p = jnp.zeros((2, _NPAD, _B), jnp.float32)
    for k in range(6):
        ab = jnp.float32(1.0) if k == 0 else alphas[k - 1]
        avec = jnp.stack([jnp.full((16,), ab, jnp.float32),
                          jnp.full((16,), 1.0 - ab, jnp.float32)])
        p = _step_call(h0_t, p, avec, rows_all, cols_all, vals_all)

    a5 = alphas[5]
    par = jnp.stack([a5, 1.0 - a5])
    bias_b = cell_emb[cell_idx] @ W_out[:, 0] + b_out[0]
    bias128 = jnp.tile(bias_b, 8)[None, :]
    y_f = _mlp_call(ctl_t.reshape(_F, 128), h0_t.reshape(_F, 128),
                    p[0].reshape(_F, 128), p[1].reshape(_F, 128),
                    W_in, b_in, W_out[:, 0], par, bias128)
    return y_f.reshape(_NPAD, _B)[:_N].T


# speedup vs baseline: 22.8981x; 1.5503x over previous
import functools

import jax
import jax.numpy as jnp
from jax import lax
from jax.experimental import pallas as pl
from jax.experimental.pallas import tpu as pltpu
from jax.experimental.pallas import tpu_sc as plsc

_N = 50000
_B = 16
_NPAD = 51200
_RPT = _NPAD // 16
_CB = 200
_NCB = _RPT // _CB
_E6 = 6 * 800000
_SUP = 512
_W = 128
_K = _SUP // _W
_NT = 32
_EPAD = ((_E6 + _NT * _SUP - 1) // (_NT * _SUP)) * (_NT * _SUP)
_EPT = _EPAD // _NT
_NSUP = _EPT // _SUP
_F = _NPAD * _B // 128
_MB = 1600


def _step_body(h0_hbm, pprev_hbm, avec_hbm, rows_hbm, cols_hbm, vals_hbm,
               pout_hbm, h_s, acc_s, gath0, gath1, rows_a, rows_b, cols_a,
               cols_b, vals_a, vals_b, avv, sem_l, sem_g, sem_s):
    gath = [gath0, gath1]
    rows = [rows_a, rows_b]
    cols = [cols_a, cols_b]
    vals = [vals_a, vals_b]
    c = lax.axis_index("c")
    s = lax.axis_index("s")
    t = c * 16 + s
    row0 = s * _RPT

    b_h0 = gath0.at[pl.ds(0, _CB)]
    b_p0 = gath0.at[pl.ds(_CB, _CB)]
    b_p1 = gath1.at[pl.ds(0, _CB)]
    b_h = gath1.at[pl.ds(_CB, _CB)]

    pltpu.sync_copy(avec_hbm, avv)
    av = avv[0, :]
    amv = avv[1, :]
    for cb in range(_NCB):
        base = row0 + cb * _CB
        pltpu.sync_copy(h0_hbm.at[pl.ds(base, _CB)], b_h0)
        pltpu.sync_copy(pprev_hbm.at[0, pl.ds(base, _CB)], b_p0)
        pltpu.sync_copy(pprev_hbm.at[1, pl.ds(base, _CB)], b_p1)

        def blend(i, _):
            b_h[i, :] = b_h0[i, :] * av + (b_p0[i, :] + b_p1[i, :]) * amv
            return 0

        lax.fori_loop(0, _CB, blend, 0, unroll=8)
        pltpu.sync_copy(b_h, h_s.at[pl.ds(base, _CB)])

    def zero(i, _):
        b_h[i, :] = jnp.zeros((16,), jnp.float32)
        return 0

    lax.fori_loop(0, _CB, zero, 0, unroll=8)
    for cb in range(_NCB):
        pltpu.sync_copy(b_h, acc_s.at[pl.ds(row0 + cb * _CB, _CB)])
    plsc.subcore_barrier()

    wbase0 = t * (_EPT // _W)
    ebase0 = t * _EPT

    def fire_lin(i, b):
        pltpu.async_copy(rows_hbm.at[pl.ds(wbase0 + i * _K, _K)],
                         rows[b], sem_l)
        pltpu.async_copy(cols_hbm.at[pl.ds(wbase0 + i * _K, _K)],
                         cols[b], sem_l)
        pltpu.async_copy(vals_hbm.at[pl.ds(ebase0 + i * _SUP, _SUP)],
                         vals[b], sem_l)

    def wait_lin(i, b):
        pltpu.make_async_copy(rows_hbm.at[pl.ds(wbase0 + i * _K, _K)],
                              rows[b], sem_l).wait()
        pltpu.make_async_copy(cols_hbm.at[pl.ds(wbase0 + i * _K, _K)],
                              cols[b], sem_l).wait()
        pltpu.make_async_copy(vals_hbm.at[pl.ds(ebase0 + i * _SUP, _SUP)],
                              vals[b], sem_l).wait()

    def fire_gath(b):
        for j in range(_K):
            pltpu.async_copy(h_s.at[rows[b].at[j]],
                             gath[b].at[pl.ds(j * _W, _W)], sem_g)

    def fire_scat(b):
        for j in range(_K):
            pltpu.async_copy(gath[b].at[pl.ds(j * _W, _W)],
                             acc_s.at[cols[b].at[j]], sem_s, add=True)

    def drain(b, sem):
        for j in range(_K):
            pltpu.make_async_copy(gath[b].at[pl.ds(j * _W, _W)],
                                  acc_s.at[cols[b].at[j]], sem).wait()

    def scale_chunk(b):
        g = gath[b]
        v = vals[b]

        def scale(gi, _):
            base = gi * 16
            v16 = v[pl.ds(base, 16)]
            for l in range(16):
                g[base + l, :] = g[base + l, :] * v16[l]
            return 0

        lax.fori_loop(0, _SUP // 16, scale, 0)

    def process(i, b):
        drain(b, sem_g)
        scale_chunk(b)
        fire_scat(b)

    fire_lin(0, 0)
    wait_lin(0, 0)
    fire_gath(0)
    fire_lin(1, 1)
    process(0, 0)
    wait_lin(1, 1)
    fire_gath(1)

    def pair(pi, _):
        i = 1 + 2 * pi
        for b in (1, 0):
            drain(1 - b, sem_s)
            fire_lin(i + 1, 1 - b)
            process(i, b)
            wait_lin(i + 1, 1 - b)
            fire_gath(1 - b)
            i = i + 1
        return 0

    lax.fori_loop(0, (_NSUP - 2) // 2, pair, 0)

    drain(0, sem_s)
    process(_NSUP - 1, 1)
    drain(1, sem_s)
    plsc.subcore_barrier()

    pltpu.sync_copy(acc_s.at[pl.ds(row0, _RPT)],
                    pout_hbm.at[c, pl.ds(row0, _RPT)])


_step_call = pl.kernel(
    _step_body,
    out_type=jax.ShapeDtypeStruct((2, _NPAD, _B), jnp.float32),
    mesh=plsc.VectorSubcoreMesh(core_axis_name="c", subcore_axis_name="s"),
    compiler_params=pltpu.CompilerParams(use_tc_tiling_on_sc=False),
    scratch_types=[
        pltpu.VMEM_SHARED((_NPAD, _B), jnp.float32),
        pltpu.VMEM_SHARED((_NPAD, _B), jnp.float32),
        pltpu.VMEM((_SUP, _B), jnp.float32),
        pltpu.VMEM((_SUP, _B), jnp.float32),
        pltpu.VMEM((_K, _W), jnp.int32),
        pltpu.VMEM((_K, _W), jnp.int32),
        pltpu.VMEM((_K, _W), jnp.int32),
        pltpu.VMEM((_K, _W), jnp.int32),
        pltpu.VMEM((_SUP,), jnp.float32),
        pltpu.VMEM((_SUP,), jnp.float32),
        pltpu.VMEM((2, 16), jnp.float32),
        pltpu.SemaphoreType.DMA,
        pltpu.SemaphoreType.DMA,
        pltpu.SemaphoreType.DMA,
    ],
    name="prop_step_sc",
)


def _mlp_body(ctl_ref, u_ref, p0_ref, p1_ref, win_ref, bin_ref, wout_ref,
              par_ref, bias_ref, out_ref):
    a5 = par_ref[0]
    am5 = par_ref[1]
    ctl = ctl_ref[...]
    u = u_ref[...]
    h = u * a5 + (p0_ref[...] + p1_ref[...]) * am5
    acc = jnp.zeros_like(ctl)
    for j in range(64):
        hh = jnp.maximum(
            ctl * win_ref[0, j] + u * win_ref[1, j] + h * win_ref[2, j]
            + bin_ref[j], 0.0)
        acc = acc + hh * wout_ref[j]
    out_ref[...] = acc + bias_ref[...]


_mlp_call = pl.pallas_call(
    _mlp_body,
    out_shape=jax.ShapeDtypeStruct((_F, 128), jnp.float32),
    grid=(_F // _MB,),
    in_specs=[
        pl.BlockSpec((_MB, 128), lambda i: (i, 0)),
        pl.BlockSpec((_MB, 128), lambda i: (i, 0)),
        pl.BlockSpec((_MB, 128), lambda i: (i, 0)),
        pl.BlockSpec((_MB, 128), lambda i: (i, 0)),
        pl.BlockSpec(memory_space=pltpu.SMEM),
        pl.BlockSpec(memory_space=pltpu.SMEM),
        pl.BlockSpec(memory_space=pltpu.SMEM),
        pl.BlockSpec(memory_space=pltpu.SMEM),
        pl.BlockSpec((1, 128), lambda i: (0, 0)),
    ],
    out_specs=pl.BlockSpec((_MB, 128), lambda i: (i, 0)),
    name="mlp_head_tc",
)


def _pad_t(x):
    return jnp.pad(x.T, ((0, _NPAD - _N), (0, 0)))


def kernel(ctl_base, u_raw, cell_idx, rows_tfp, cols_tfp, vals_tfp,
           rows_tfn, cols_tfn, vals_tfn, rows_ppp, cols_ppp, vals_ppp,
           rows_ppn, cols_ppn, vals_ppn, rows_und, cols_und, vals_und,
           rows_mir, cols_mir, vals_mir, g_tf_pos, g_tf_neg, g_ppi_pos,
           g_ppi_neg, g_undir, g_mirna_neg, alpha_logits, cell_emb,
           W_in, b_in, W_out, b_out):
    sp = jax.nn.softplus
    gains = [sp(g_tf_pos), -sp(g_tf_neg), sp(g_ppi_pos), -sp(g_ppi_neg),
             sp(g_undir), -sp(g_mirna_neg)]
    alphas = jax.nn.sigmoid(alpha_logits)

    npad_e = _EPAD - _E6
    pad_idx = (jnp.arange(npad_e, dtype=jnp.int32) % _N)
    rows_all = jnp.concatenate(
        [rows_tfp, rows_tfn, rows_ppp, rows_ppn, rows_und, rows_mir,
         pad_idx]).reshape(_EPAD // _W, _W)
    cols_all = jnp.concatenate(
        [cols_tfp, cols_tfn, cols_ppp, cols_ppn, cols_und, cols_mir,
         pad_idx]).reshape(_EPAD // _W, _W)
    vals_all = jnp.concatenate(
        [g * v for g, v in zip(gains, [vals_tfp, vals_tfn, vals_ppp,
                                       vals_ppn, vals_und, vals_mir])]
        + [jnp.zeros((npad_e,), jnp.float32)])

    h0_t = _pad_t(u_raw)
    ctl_t = _pad_t(ctl_base)

    p = jnp.zeros((2, _NPAD, _B), jnp.float32)
    for k in range(6):
        ab = jnp.float32(1.0) if k == 0 else alphas[k - 1]
        avec = jnp.stack([jnp.full((16,), ab, jnp.float32),
                          jnp.full((16,), 1.0 - ab, jnp.float32)])
        p = _step_call(h0_t, p, avec, rows_all, cols_all, vals_all)

    a5 = alphas[5]
    par = jnp.stack([a5, 1.0 - a5])
    bias_b = cell_emb[cell_idx] @ W_out[:, 0] + b_out[0]
    bias128 = jnp.tile(bias_b, 8)[None, :]
    y_f = _mlp_call(ctl_t.reshape(_F, 128), h0_t.reshape(_F, 128),
                    p[0].reshape(_F, 128), p[1].reshape(_F, 128),
                    W_in, b_in, W_out[:, 0], par, bias128)
    return y_f.reshape(_NPAD, _B)[:_N].T


# speedup vs baseline: 23.3765x; 1.0209x over previous
import functools

import jax
import jax.numpy as jnp
from jax import lax
from jax.experimental import pallas as pl
from jax.experimental.pallas import tpu as pltpu
from jax.experimental.pallas import tpu_sc as plsc

_N = 50000
_B = 16
_NPAD = 51200
_RPT = _NPAD // 16
_CB = 200
_NCB = _RPT // _CB
_E6 = 6 * 800000
_SUP = 512
_W = 128
_K = _SUP // _W
_NT = 32
_EPAD = ((_E6 + _NT * _SUP - 1) // (_NT * _SUP)) * (_NT * _SUP)
_EPT = _EPAD // _NT
_NSUP = _EPT // _SUP
_STEPS = 6
_F = _NPAD * _B // 128
_MB = 1600


def _prop_body(h0_hbm, avec_hbm, rows_hbm, cols_hbm, vals_hbm, eflag_hbm,
               bflag_hbm, pout_hbm, h_s, acc_s, gath0, gath1, rows_a,
               rows_b, cols_a, cols_b, vals_a, vals_b, avv, mark_v,
               probe_v, done_v, sem_l, sem_g, sem_s):
    gath = [gath0, gath1]
    rows = [rows_a, rows_b]
    cols = [cols_a, cols_b]
    vals = [vals_a, vals_b]
    c = lax.axis_index("c")
    s = lax.axis_index("s")
    oc = 1 - c
    t = c * 16 + s
    row0 = s * _RPT
    wbase0 = t * (_EPT // _W)
    ebase0 = t * _EPT

    b_h0 = gath0.at[pl.ds(0, _CB)]
    b_p0 = gath0.at[pl.ds(_CB, _CB)]
    b_p1 = gath1.at[pl.ds(0, _CB)]
    b_h = gath1.at[pl.ds(_CB, _CB)]

    mark_v[...] = jnp.ones((16,), jnp.float32)
    pltpu.sync_copy(avec_hbm, avv)

    def poll(flag_slice):
        done_v[0] = 0

        def outer(_, __):
            @pl.when(done_v[0] == 0)
            def _():
                def inner(_, __):
                    @pl.when(done_v[0] == 0)
                    def _():
                        pltpu.sync_copy(flag_slice, probe_v)
                        done_v[0] = (jnp.sum(probe_v[...])
                                     != 0.0).astype(jnp.int32)
                    return 0

                lax.fori_loop(0, 32, inner, 0)
            return 0

        lax.fori_loop(0, 256, outer, 0)

    def fire_lin(i, b):
        pltpu.async_copy(rows_hbm.at[pl.ds(wbase0 + i * _K, _K)],
                         rows[b], sem_l)
        pltpu.async_copy(cols_hbm.at[pl.ds(wbase0 + i * _K, _K)],
                         cols[b], sem_l)
        pltpu.async_copy(vals_hbm.at[pl.ds(ebase0 + i * _SUP, _SUP)],
                         vals[b], sem_l)

    def wait_lin(i, b):
        pltpu.make_async_copy(rows_hbm.at[pl.ds(wbase0 + i * _K, _K)],
                              rows[b], sem_l).wait()
        pltpu.make_async_copy(cols_hbm.at[pl.ds(wbase0 + i * _K, _K)],
                              cols[b], sem_l).wait()
        pltpu.make_async_copy(vals_hbm.at[pl.ds(ebase0 + i * _SUP, _SUP)],
                              vals[b], sem_l).wait()

    def fire_gath(b):
        for j in range(_K):
            pltpu.async_copy(h_s.at[rows[b].at[j]],
                             gath[b].at[pl.ds(j * _W, _W)], sem_g)

    def fire_scat(b):
        for j in range(_K):
            pltpu.async_copy(gath[b].at[pl.ds(j * _W, _W)],
                             acc_s.at[cols[b].at[j]], sem_s, add=True)

    def drain(b, sem):
        for j in range(_K):
            pltpu.make_async_copy(gath[b].at[pl.ds(j * _W, _W)],
                                  acc_s.at[cols[b].at[j]], sem).wait()

    def scale_chunk(b):
        g = gath[b]
        v = vals[b]

        def scale(gi, _):
            base = gi * 16
            v16 = v[pl.ds(base, 16)]
            for l in range(16):
                g[base + l, :] = g[base + l, :] * v16[l]
            return 0

        lax.fori_loop(0, _SUP // 16, scale, 0)

    def process(b):
        drain(b, sem_g)
        scale_chunk(b)
        fire_scat(b)

    def step(k, _):
        @pl.when(k == 0)
        def _():
            pltpu.sync_copy(h0_hbm.at[pl.ds(row0, _RPT)],
                            h_s.at[pl.ds(row0, _RPT)])

        @pl.when(k > 0)
        def _():
            poll(eflag_hbm.at[oc, k - 1])
            av = avv[2 * k, :]
            amv = avv[2 * k + 1, :]
            for cb in range(_NCB):
                base = row0 + cb * _CB
                pltpu.sync_copy(h0_hbm.at[pl.ds(base, _CB)], b_h0)
                pltpu.sync_copy(pout_hbm.at[0, pl.ds(base, _CB)], b_p0)
                pltpu.sync_copy(pout_hbm.at[1, pl.ds(base, _CB)], b_p1)

                def blend(i, _):
                    b_h[i, :] = (b_h0[i, :] * av
                                 + (b_p0[i, :] + b_p1[i, :]) * amv)
                    return 0

                lax.fori_loop(0, _CB, blend, 0, unroll=8)
                pltpu.sync_copy(b_h, h_s.at[pl.ds(base, _CB)])
            plsc.subcore_barrier()

            @pl.when(s == 0)
            def _():
                pltpu.sync_copy(mark_v, bflag_hbm.at[c, k])

        def zero(i, _):
            b_h[i, :] = jnp.zeros((16,), jnp.float32)
            return 0

        lax.fori_loop(0, _CB, zero, 0, unroll=8)
        for cb in range(_NCB):
            pltpu.sync_copy(b_h, acc_s.at[pl.ds(row0 + cb * _CB, _CB)])
        plsc.subcore_barrier()

        fire_lin(0, 0)
        wait_lin(0, 0)
        fire_gath(0)
        fire_lin(1, 1)
        process(0)
        wait_lin(1, 1)
        fire_gath(1)

        def pair(pi, _):
            i = 1 + 2 * pi
            for b in (1, 0):
                drain(1 - b, sem_s)
                fire_lin(i + 1, 1 - b)
                process(b)
                wait_lin(i + 1, 1 - b)
                fire_gath(1 - b)
                i = i + 1
            return 0

        lax.fori_loop(0, (_NSUP - 2) // 2, pair, 0)

        drain(0, sem_s)
        process(1)
        drain(1, sem_s)
        plsc.subcore_barrier()

        @pl.when(k > 0)
        def _():
            poll(bflag_hbm.at[oc, k])

        pltpu.sync_copy(acc_s.at[pl.ds(row0, _RPT)],
                        pout_hbm.at[c, pl.ds(row0, _RPT)])
        plsc.subcore_barrier()

        @pl.when(s == 0)
        def _():
            pltpu.sync_copy(mark_v, eflag_hbm.at[c, k])

        return 0

    lax.fori_loop(0, _STEPS, step, 0)


_prop_call = pl.kernel(
    _prop_body,
    out_type=jax.ShapeDtypeStruct((2, _NPAD, _B), jnp.float32),
    mesh=plsc.VectorSubcoreMesh(core_axis_name="c", subcore_axis_name="s"),
    compiler_params=pltpu.CompilerParams(use_tc_tiling_on_sc=False,
                                         needs_layout_passes=False),
    scratch_types=[
        pltpu.VMEM_SHARED((_NPAD, _B), jnp.float32),
        pltpu.VMEM_SHARED((_NPAD, _B), jnp.float32),
        pltpu.VMEM((_SUP, _B), jnp.float32),
        pltpu.VMEM((_SUP, _B), jnp.float32),
        pltpu.VMEM((_K, _W), jnp.int32),
        pltpu.VMEM((_K, _W), jnp.int32),
        pltpu.VMEM((_K, _W), jnp.int32),
        pltpu.VMEM((_K, _W), jnp.int32),
        pltpu.VMEM((_SUP,), jnp.float32),
        pltpu.VMEM((_SUP,), jnp.float32),
        pltpu.VMEM((2 * _STEPS, 16), jnp.float32),
        pltpu.VMEM((16,), jnp.float32),
        pltpu.VMEM((16,), jnp.float32),
        pltpu.SMEM((1,), jnp.int32),
        pltpu.SemaphoreType.DMA,
        pltpu.SemaphoreType.DMA,
        pltpu.SemaphoreType.DMA,
    ],
    name="prop_sc",
)


def _zero_body(e_ref, b_ref):
    e_ref[...] = jnp.zeros_like(e_ref)
    b_ref[...] = jnp.zeros_like(b_ref)


_zero_call = pl.pallas_call(
    _zero_body,
    out_shape=[jax.ShapeDtypeStruct((2, _STEPS, 16), jnp.float32),
               jax.ShapeDtypeStruct((2, _STEPS, 16), jnp.float32)],
    name="zero_flags_tc",
)


def _mlp_body(ctl_ref, u_ref, p0_ref, p1_ref, win_ref, bin_ref, wout_ref,
              par_ref, bias_ref, out_ref):
    a5 = par_ref[0]
    am5 = par_ref[1]
    ctl = ctl_ref[...]
    u = u_ref[...]
    h = u * a5 + (p0_ref[...] + p1_ref[...]) * am5
    acc = jnp.zeros_like(ctl)
    for j in range(64):
        hh = jnp.maximum(
            ctl * win_ref[0, j] + u * win_ref[1, j] + h * win_ref[2, j]
            + bin_ref[j], 0.0)
        acc = acc + hh * wout_ref[j]
    out_ref[...] = acc + bias_ref[...]


_mlp_call = pl.pallas_call(
    _mlp_body,
    out_shape=jax.ShapeDtypeStruct((_F, 128), jnp.float32),
    grid=(_F // _MB,),
    in_specs=[
        pl.BlockSpec((_MB, 128), lambda i: (i, 0)),
        pl.BlockSpec((_MB, 128), lambda i: (i, 0)),
        pl.BlockSpec((_MB, 128), lambda i: (i, 0)),
        pl.BlockSpec((_MB, 128), lambda i: (i, 0)),
        pl.BlockSpec(memory_space=pltpu.SMEM),
        pl.BlockSpec(memory_space=pltpu.SMEM),
        pl.BlockSpec(memory_space=pltpu.SMEM),
        pl.BlockSpec(memory_space=pltpu.SMEM),
        pl.BlockSpec((1, 128), lambda i: (0, 0)),
    ],
    out_specs=pl.BlockSpec((_MB, 128), lambda i: (i, 0)),
    name="mlp_head_tc",
)


def _pad_t(x):
    return jnp.pad(x.T, ((0, _NPAD - _N), (0, 0)))


def kernel(ctl_base, u_raw, cell_idx, rows_tfp, cols_tfp, vals_tfp,
           rows_tfn, cols_tfn, vals_tfn, rows_ppp, cols_ppp, vals_ppp,
           rows_ppn, cols_ppn, vals_ppn, rows_und, cols_und, vals_und,
           rows_mir, cols_mir, vals_mir, g_tf_pos, g_tf_neg, g_ppi_pos,
           g_ppi_neg, g_undir, g_mirna_neg, alpha_logits, cell_emb,
           W_in, b_in, W_out, b_out):
    sp = jax.nn.softplus
    gains = [sp(g_tf_pos), -sp(g_tf_neg), sp(g_ppi_pos), -sp(g_ppi_neg),
             sp(g_undir), -sp(g_mirna_neg)]
    alphas = jax.nn.sigmoid(alpha_logits)

    npad_e = _EPAD - _E6
    pad_idx = (jnp.arange(npad_e, dtype=jnp.int32) % _N)
    rows_all = jnp.concatenate(
        [rows_tfp, rows_tfn, rows_ppp, rows_ppn, rows_und, rows_mir,
         pad_idx]).reshape(_EPAD // _W, _W)
    cols_all = jnp.concatenate(
        [cols_tfp, cols_tfn, cols_ppp, cols_ppn, cols_und, cols_mir,
         pad_idx]).reshape(_EPAD // _W, _W)
    vals_all = jnp.concatenate(
        [g * v for g, v in zip(gains, [vals_tfp, vals_tfn, vals_ppp,
                                       vals_ppn, vals_und, vals_mir])]
        + [jnp.zeros((npad_e,), jnp.float32)])

    h0_t = _pad_t(u_raw)
    ctl_t = _pad_t(ctl_base)

    ab = jnp.concatenate([jnp.ones((1,), jnp.float32), alphas[:_STEPS - 1]])
    avec = jnp.stack([jnp.stack([jnp.full((16,), ab[k], jnp.float32),
                                 jnp.full((16,), 1.0 - ab[k], jnp.float32)])
                      for k in range(_STEPS)]).reshape(2 * _STEPS, 16)

    eflag, bflag = _zero_call()
    p = _prop_call(h0_t, avec, rows_all, cols_all, vals_all, eflag, bflag)

    a5 = alphas[5]
    par = jnp.stack([a5, 1.0 - a5])
    bias_b = cell_emb[cell_idx] @ W_out[:, 0] + b_out[0]
    bias128 = jnp.tile(bias_b, 8)[None, :]
    y_f = _mlp_call(ctl_t.reshape(_F, 128), h0_t.reshape(_F, 128),
                    p[0].reshape(_F, 128), p[1].reshape(_F, 128),
                    W_in, b_in, W_out[:, 0], par, bias128)
    return y_f.reshape(_NPAD, _B)[:_N].T
